# baseline (speedup 1.0000x reference)
import functools

import jax
import jax.numpy as jnp
from jax.experimental import pallas as pl
from jax.experimental.pallas import tpu as pltpu

_ROWS = 256
_COLS = 256
_NEDGE = 64
_CHUNK = 64


def _dice_kernel(edges_ref, dmap_ref, out_ref):
    s = pl.program_id(0)

    e = jnp.clip(edges_ref[0] * 255.0, 0.0, 255.0)
    px = e[0:1, :]
    py = e[1:2, :]
    pjx = e[2:3, :]
    pjy = e[3:4, :]

    jj = jax.lax.broadcasted_iota(jnp.int32, (_COLS, _COLS), 0)
    xx = jax.lax.broadcasted_iota(jnp.int32, (_COLS, _COLS), 1)
    l_incl = (jj <= xx).astype(jnp.bfloat16)
    l_strict = (jj < xx).astype(jnp.bfloat16)

    inter = 0.0
    s_true = 0.0
    s_pred = 0.0
    for c in range(_ROWS // _CHUNK):
        ys = (jax.lax.broadcasted_iota(jnp.int32, (_CHUNK, 1), 0)
              .astype(jnp.float32) + float(c * _CHUNK))
        cond = ((py < ys) & (pjy >= ys)) | ((pjy < ys) & (py >= ys))
        dy = pjy - py
        denom = jnp.where(dy == 0.0, 1.0, dy)
        xc = px + (ys - py) / denom * (pjx - px)
        fc = jnp.floor(jnp.clip(xc, 0.0, 255.0))

        fcm = jnp.where(cond, fc, 400.0).astype(jnp.bfloat16)
        jbins = (jax.lax.broadcasted_iota(jnp.int32, (_CHUNK, _NEDGE, _COLS), 2)
                 .astype(jnp.bfloat16))
        fcm3 = jax.lax.broadcast_in_dim(fcm, (_CHUNK, _NEDGE, _COLS), (0, 1))
        hist = jnp.sum((fcm3 == jbins).astype(jnp.bfloat16),
                       axis=1)
        a = jax.lax.dot(hist, l_incl, preferred_element_type=jnp.float32)
        b = jax.lax.dot(hist, l_strict, preferred_element_type=jnp.float32)
        r_tot = a[:, _COLS - 1:_COLS]
        m_lim = r_tot - 1.0 - (r_tot - 2.0 * jnp.floor(r_tot * 0.5))
        b_odd = b - 2.0 * jnp.floor(b * 0.5)
        f_odd = (b <= m_lim).astype(jnp.float32)
        f_even = ((a > b).astype(jnp.float32)
                  * ((b + 1.0) <= m_lim).astype(jnp.float32))
        filled = b_odd * f_odd + (1.0 - b_odd) * f_even

        dchunk = dmap_ref[0, c * _CHUNK:(c + 1) * _CHUNK, :]
        binary = (dchunk * 255.0 <= 127.0).astype(jnp.float32)

        inter = inter + jnp.sum(filled * binary)
        s_true = s_true + jnp.sum(filled)
        s_pred = s_pred + jnp.sum(binary)

    smooth = 1e-06
    dice = (2.0 * inter + smooth) / (s_true + s_pred + smooth)
    loss = 1.0 - dice

    @pl.when(s == 0)
    def _init():
        out_ref[...] = jnp.zeros_like(out_ref)

    out_ref[...] += loss * (1.0 / 64.0)


@jax.jit
def _run(edges, dmap):
    nsam = edges.shape[0]
    out = pl.pallas_call(
        _dice_kernel,
        grid=(nsam,),
        in_specs=[
            pl.BlockSpec((1, 8, _NEDGE), lambda s: (s, 0, 0)),
            pl.BlockSpec((1, _ROWS, _COLS), lambda s: (s, 0, 0)),
        ],
        out_specs=pl.BlockSpec((8, 128), lambda s: (0, 0)),
        out_shape=jax.ShapeDtypeStruct((8, 128), jnp.float32),
        compiler_params=pltpu.CompilerParams(
            dimension_semantics=("arbitrary",),
        ),
    )(edges, dmap)
    return out[0, 0]


def kernel(points, distance_map):
    pts = points[:, :, 0, :]
    px = pts[:, :, 0]
    py = pts[:, :, 1]
    pjx = jnp.roll(px, 1, axis=1)
    pjy = jnp.roll(py, 1, axis=1)
    zeros = jnp.zeros_like(px)
    edges = jnp.stack([px, py, pjx, pjy, zeros, zeros, zeros, zeros],
                      axis=1)
    dmap = distance_map[:, :, :, 0]
    return _run(edges, dmap)


# speedup vs baseline: 2.0504x; 2.0504x over previous
import functools

import jax
import jax.numpy as jnp
from jax import lax
from jax.experimental import pallas as pl
from jax.experimental.pallas import tpu as pltpu
from jax.experimental.pallas import tpu_sc as plsc

_ROWS = 256
_COLS = 256
_NEDGE = 64
_NSAM = 64
_NC = 2
_NS = 16
_NW = _NC * _NS
_RG = 16
_NTASK = _NSAM * (_ROWS // _RG)
_TPW = _NTASK // _NW


def _raster_sc(pts_hbm, zeros_hbm, hist_hbm, pts_v, hist_v):
    wid = lax.axis_index("s") * _NC + lax.axis_index("c")
    lane = lax.broadcasted_iota(jnp.int32, (_RG,), 0)
    lane_f = lane.astype(jnp.float32)
    ones = jnp.ones((_RG,), jnp.float32)

    def task(i, carry):
        g = wid * _TPW + i
        sample = g // (_ROWS // _RG)
        rg = g - sample * (_ROWS // _RG)

        @pl.when(rg == 0)
        def _load_pts():
            pltpu.sync_copy(pts_hbm.at[sample], pts_v)

        pltpu.sync_copy(zeros_hbm, hist_v)

        chunks = [jnp.clip(pts_v[pl.ds(c * _RG, _RG)] * 255.0, 0.0, 255.0)
                  for c in range(4 * _NEDGE // _RG)]

        ys = lane_f + (rg * _RG).astype(jnp.float32)
        for e in range(_NEDGE):
            c, j = e // _RG, e % _RG
            idx_j = jnp.full((_RG,), j, jnp.int32)
            px = chunks[c].at[idx_j].get(mode="promise_in_bounds")
            py = chunks[4 + c].at[idx_j].get(mode="promise_in_bounds")
            pjx = chunks[8 + c].at[idx_j].get(mode="promise_in_bounds")
            pjy = chunks[12 + c].at[idx_j].get(mode="promise_in_bounds")
            cond = ((py < ys) & (pjy >= ys)) | ((pjy < ys) & (py >= ys))
            dy = pjy - py
            denom = jnp.where(dy == 0.0, 1.0, dy)
            q = (ys - py) / denom
            xc = px + q * (pjx - px)
            bins = jnp.clip(xc, 0.0, 255.0).astype(jnp.int32)
            plsc.addupdate_scatter(hist_v, [lane, bins], ones, mask=cond)

        pltpu.sync_copy(hist_v, hist_hbm.at[sample, pl.ds(rg * _RG, _RG)])
        return carry

    lax.fori_loop(0, _TPW, task, 0)


def _dice_tc(hist_ref, dmap_ref, out_ref):
    s = pl.program_id(0)

    jj = jax.lax.broadcasted_iota(jnp.int32, (_COLS, _COLS), 0)
    xx = jax.lax.broadcasted_iota(jnp.int32, (_COLS, _COLS), 1)
    l_incl = (jj <= xx).astype(jnp.float32)

    hist = hist_ref[0]
    a = jax.lax.dot(hist, l_incl, preferred_element_type=jnp.float32)
    b = a - hist
    r_tot = a[:, _COLS - 1:_COLS]
    m_lim = r_tot - 1.0 - (r_tot - 2.0 * jnp.floor(r_tot * 0.5))
    b_odd = b - 2.0 * jnp.floor(b * 0.5)
    f_odd = (b <= m_lim).astype(jnp.float32)
    f_even = ((a > b).astype(jnp.float32)
              * ((b + 1.0) <= m_lim).astype(jnp.float32))
    filled = b_odd * f_odd + (1.0 - b_odd) * f_even

    binary = (dmap_ref[0] * 255.0 <= 127.0).astype(jnp.float32)

    inter = jnp.sum(filled * binary)
    s_true = jnp.sum(filled)
    s_pred = jnp.sum(binary)

    smooth = 1e-06
    loss = 1.0 - (2.0 * inter + smooth) / (s_true + s_pred + smooth)

    @pl.when(s == 0)
    def _init():
        out_ref[...] = jnp.zeros_like(out_ref)

    out_ref[...] += loss * (1.0 / _NSAM)


@jax.jit
def _run(pts_sc, dmap):
    zeros = jnp.zeros((_RG, _COLS), jnp.float32)
    raster = pl.kernel(
        _raster_sc,
        out_type=jax.ShapeDtypeStruct((_NSAM, _ROWS, _COLS), jnp.float32),
        mesh=plsc.VectorSubcoreMesh(core_axis_name="c", subcore_axis_name="s",
                                    num_cores=_NC, num_subcores=_NS),
        scratch_types=[
            pltpu.VMEM((4 * _NEDGE,), jnp.float32),
            pltpu.VMEM((_RG, _COLS), jnp.float32),
        ],
        compiler_params=pltpu.CompilerParams(use_tc_tiling_on_sc=False,
                                             needs_layout_passes=False),
    )
    hist = raster(pts_sc, zeros)

    out = pl.pallas_call(
        _dice_tc,
        grid=(_NSAM,),
        in_specs=[
            pl.BlockSpec((1, _ROWS, _COLS), lambda s: (s, 0, 0)),
            pl.BlockSpec((1, _ROWS, _COLS), lambda s: (s, 0, 0)),
        ],
        out_specs=pl.BlockSpec((8, 128), lambda s: (0, 0)),
        out_shape=jax.ShapeDtypeStruct((8, 128), jnp.float32),
        compiler_params=pltpu.CompilerParams(
            dimension_semantics=("arbitrary",),
        ),
    )(hist, dmap)
    return out[0, 0]


def kernel(points, distance_map):
    pts = points[:, :, 0, :]
    px = pts[:, :, 0]
    py = pts[:, :, 1]
    pjx = jnp.roll(px, 1, axis=1)
    pjy = jnp.roll(py, 1, axis=1)
    pts_sc = jnp.concatenate([px, py, pjx, pjy], axis=1)
    dmap = distance_map[:, :, :, 0]
    return _run(pts_sc, dmap)
